# Initial kernel scaffold; baseline (speedup 1.0000x reference)
import jax
import jax.numpy as jnp
from jax.experimental import pallas as pl


def kernel(edge_index, edge_attr, batch, W1, b1, W2, b2, W3, b3):
    raise NotImplementedError("write your pallas kernel here")


# speedup vs baseline: 24.6780x; 24.6780x over previous
import functools

import jax
import jax.numpy as jnp
from jax.experimental import pallas as pl
from jax.experimental.pallas import tpu as pltpu

_EDGE_TILE = 6400
_NODE_CHUNK = 800


def _fused_kernel(num_tiles, num_graphs, n_nodes, node_rows,
                  src_ref, attr_ref, batch_ref, w1_ref, b1_ref, w2_ref,
                  b2_ref, w3_ref, b3_ref, out_ref,
                  starts_ref, ends_ref, acc_ref, cnt_ref):
    i = pl.program_id(0)

    @pl.when(i == 0)
    def _init():
        gcol = jax.lax.broadcasted_iota(jnp.int32, (num_graphs, 1), 0)

        def body(j, carry):
            s, e = carry
            row = batch_ref[pl.ds(j, 1), :]
            s = s + jnp.sum((gcol > row).astype(jnp.int32), axis=1,
                            keepdims=True)
            e = e + jnp.sum((gcol >= row).astype(jnp.int32), axis=1,
                            keepdims=True)
            return s, e

        zero = jnp.zeros((num_graphs, 1), jnp.int32)
        s, e = jax.lax.fori_loop(0, node_rows, body, (zero, zero))
        starts_ref[...] = s
        ends_ref[...] = e
        acc_ref[...] = jnp.zeros_like(acc_ref)
        cnt_ref[...] = jnp.zeros_like(cnt_ref)

    node = src_ref[0]
    attr = attr_ref[...]
    h = jnp.dot(attr, w1_ref[...], preferred_element_type=jnp.float32)
    h = jnp.maximum(h + b1_ref[...], 0.0)
    onehot = jnp.logical_and(node >= starts_ref[...],
                             node < ends_ref[...]).astype(jnp.float32)
    acc_ref[...] += jax.lax.dot_general(
        onehot, h, (((1,), (0,)), ((), ())),
        preferred_element_type=jnp.float32)
    cnt_ref[...] += jnp.sum(onehot, axis=1, keepdims=True)

    @pl.when(i == num_tiles - 1)
    def _finish():
        gf = acc_ref[...] / jnp.maximum(cnt_ref[...], 1.0)
        h2 = jnp.dot(gf, w2_ref[...], preferred_element_type=jnp.float32)
        h2 = jnp.maximum(h2 + b2_ref[...], 0.0)
        out = jnp.dot(h2, w3_ref[...], preferred_element_type=jnp.float32)
        out_ref[...] = out + b3_ref[...]


@jax.jit
def kernel(edge_index, edge_attr, batch, W1, b1, W2, b2, W3, b3):
    n_edges, in_dim = edge_attr.shape
    n_nodes = batch.shape[0]
    hidden = W1.shape[1]
    out_dim = W3.shape[1]
    num_graphs = 128

    et = _EDGE_TILE
    pad_e = (-n_edges) % et
    src = edge_index[0].astype(jnp.int32)
    attr = edge_attr
    if pad_e:
        src = jnp.concatenate(
            [src, jnp.full((pad_e,), n_nodes, jnp.int32)])
        attr = jnp.concatenate(
            [attr, jnp.zeros((pad_e, in_dim), attr.dtype)])
    num_tiles = (n_edges + pad_e) // et
    src = src.reshape(num_tiles, 1, et)

    b32 = batch.astype(jnp.int32)
    pad_n = (-n_nodes) % _NODE_CHUNK
    if pad_n:
        b32 = jnp.concatenate(
            [b32, jnp.full((pad_n,), num_graphs, jnp.int32)])
    node_rows = (n_nodes + pad_n) // _NODE_CHUNK
    b32 = b32.reshape(node_rows, _NODE_CHUNK)

    grid = (num_tiles,)
    const = lambda i: (0, 0)
    out = pl.pallas_call(
        functools.partial(_fused_kernel, num_tiles, num_graphs, n_nodes,
                          node_rows),
        grid=grid,
        in_specs=[
            pl.BlockSpec((1, 1, et), lambda i: (i, 0, 0)),
            pl.BlockSpec((et, in_dim), lambda i: (i, 0)),
            pl.BlockSpec((node_rows, _NODE_CHUNK), const),
            pl.BlockSpec((in_dim, hidden), const),
            pl.BlockSpec((1, hidden), const),
            pl.BlockSpec((hidden, hidden), const),
            pl.BlockSpec((1, hidden), const),
            pl.BlockSpec((hidden, out_dim), const),
            pl.BlockSpec((1, out_dim), const),
        ],
        out_specs=pl.BlockSpec((num_graphs, out_dim), const),
        out_shape=jax.ShapeDtypeStruct((num_graphs, out_dim), jnp.float32),
        scratch_shapes=[
            pltpu.VMEM((num_graphs, 1), jnp.int32),
            pltpu.VMEM((num_graphs, 1), jnp.int32),
            pltpu.VMEM((num_graphs, hidden), jnp.float32),
            pltpu.VMEM((num_graphs, 1), jnp.float32),
        ],
        compiler_params=pltpu.CompilerParams(
            dimension_semantics=("arbitrary",)),
    )(src, attr, b32, W1, b1.reshape(1, hidden), W2,
      b2.reshape(1, hidden), W3, b3.reshape(1, out_dim))
    return out


# speedup vs baseline: 26.6546x; 1.0801x over previous
import functools

import jax
import jax.numpy as jnp
from jax.experimental import pallas as pl
from jax.experimental.pallas import tpu as pltpu

_EDGE_TILE = 6400
_NODE_CHUNK = 800
_NB = 136


def _fused_kernel(num_tiles, num_graphs, n_nodes, node_rows,
                  src_ref, attr_ref, batch_ref, w1_ref, b1_ref, w2_ref,
                  b2_ref, w3_ref, b3_ref, out_ref, starts_ref, acc_ref):
    i = pl.program_id(0)
    et = attr_ref.shape[0]

    @pl.when(i == 0)
    def _init():
        gcol = jax.lax.broadcasted_iota(jnp.int32, (_NB, 1), 0)

        def body(j, s):
            row = batch_ref[pl.ds(j, 1), :]
            return s + jnp.sum((gcol > row).astype(jnp.int32), axis=1,
                               keepdims=True)

        zero = jnp.zeros((_NB, 1), jnp.int32)
        starts_ref[...] = jax.lax.fori_loop(0, node_rows, body, zero)
        acc_ref[...] = jnp.zeros_like(acc_ref)

    node = src_ref[0]
    one = jnp.ones((), jnp.float32)
    zro = jnp.zeros((), jnp.float32)
    u = jnp.where(node >= starts_ref[...], one, zro).astype(jnp.bfloat16)
    attr16 = attr_ref[...].astype(jnp.bfloat16)
    h = jnp.dot(attr16, w1_ref[...].astype(jnp.bfloat16),
                preferred_element_type=jnp.float32)
    h = jnp.maximum(h + b1_ref[...], 0.0)
    h_aug = jnp.concatenate(
        [h.astype(jnp.bfloat16), jnp.ones((et, 1), jnp.bfloat16)], axis=1)
    p = jax.lax.dot_general(u, h_aug, (((1,), (0,)), ((), ())),
                            preferred_element_type=jnp.float32)
    acc_ref[...] += p[0:num_graphs, :] - p[1:num_graphs + 1, :]

    @pl.when(i == num_tiles - 1)
    def _finish():
        a = acc_ref[...]
        hidden = w2_ref.shape[0]
        gf = a[:, 0:hidden] / jnp.maximum(a[:, hidden:hidden + 1], 1.0)
        h2 = jnp.dot(gf, w2_ref[...], preferred_element_type=jnp.float32)
        h2 = jnp.maximum(h2 + b2_ref[...], 0.0)
        out = jnp.dot(h2, w3_ref[...], preferred_element_type=jnp.float32)
        out_ref[...] = out + b3_ref[...]


@jax.jit
def kernel(edge_index, edge_attr, batch, W1, b1, W2, b2, W3, b3):
    n_edges, in_dim = edge_attr.shape
    n_nodes = batch.shape[0]
    hidden = W1.shape[1]
    out_dim = W3.shape[1]
    num_graphs = 128

    et = _EDGE_TILE
    pad_e = (-n_edges) % et
    src = edge_index[0].astype(jnp.int32)
    attr = edge_attr
    if pad_e:
        src = jnp.concatenate(
            [src, jnp.full((pad_e,), n_nodes, jnp.int32)])
        attr = jnp.concatenate(
            [attr, jnp.zeros((pad_e, in_dim), attr.dtype)])
    num_tiles = (n_edges + pad_e) // et
    src = src.reshape(num_tiles, 1, et)

    b32 = batch.astype(jnp.int32)
    pad_n = (-n_nodes) % _NODE_CHUNK
    if pad_n:
        b32 = jnp.concatenate(
            [b32, jnp.full((pad_n,), num_graphs, jnp.int32)])
    node_rows = (n_nodes + pad_n) // _NODE_CHUNK
    b32 = b32.reshape(node_rows, _NODE_CHUNK)

    grid = (num_tiles,)
    const = lambda i: (0, 0)
    out = pl.pallas_call(
        functools.partial(_fused_kernel, num_tiles, num_graphs, n_nodes,
                          node_rows),
        grid=grid,
        in_specs=[
            pl.BlockSpec((1, 1, et), lambda i: (i, 0, 0)),
            pl.BlockSpec((et, in_dim), lambda i: (i, 0)),
            pl.BlockSpec((node_rows, _NODE_CHUNK), const),
            pl.BlockSpec((in_dim, hidden), const),
            pl.BlockSpec((1, hidden), const),
            pl.BlockSpec((hidden, hidden), const),
            pl.BlockSpec((1, hidden), const),
            pl.BlockSpec((hidden, out_dim), const),
            pl.BlockSpec((1, out_dim), const),
        ],
        out_specs=pl.BlockSpec((num_graphs, out_dim), const),
        out_shape=jax.ShapeDtypeStruct((num_graphs, out_dim), jnp.float32),
        scratch_shapes=[
            pltpu.VMEM((_NB, 1), jnp.int32),
            pltpu.VMEM((num_graphs, hidden + 1), jnp.float32),
        ],
        compiler_params=pltpu.CompilerParams(
            dimension_semantics=("arbitrary",)),
    )(src, attr, b32, W1, b1.reshape(1, hidden), W2,
      b2.reshape(1, hidden), W3, b3.reshape(1, out_dim))
    return out


# speedup vs baseline: 31.2493x; 1.1724x over previous
import functools

import jax
import jax.numpy as jnp
from jax.experimental import pallas as pl
from jax.experimental.pallas import tpu as pltpu

_EDGE_TILE = 25600
_NODE_CHUNK = 800
_NB = 136


def _fused_kernel(num_tiles, num_graphs, n_nodes, node_rows,
                  src_ref, attr_ref, batch_ref, w1_ref, b1_ref, w2_ref,
                  b2_ref, w3_ref, b3_ref, out_ref, starts_ref, acc_ref):
    i = pl.program_id(0)
    et = attr_ref.shape[0]

    @pl.when(i == 0)
    def _init():
        gcol = jax.lax.broadcasted_iota(jnp.int32, (_NB, 1), 0)

        def body(j, s):
            row = batch_ref[pl.ds(j, 1), :]
            return s + jnp.sum((gcol > row).astype(jnp.int32), axis=1,
                               keepdims=True)

        zero = jnp.zeros((_NB, 1), jnp.int32)
        starts_ref[...] = jax.lax.fori_loop(0, node_rows, body, zero)
        acc_ref[...] = jnp.zeros_like(acc_ref)

    node = src_ref[0]
    one = jnp.ones((), jnp.float32)
    zro = jnp.zeros((), jnp.float32)
    u = jnp.where(node >= starts_ref[...], one, zro).astype(jnp.bfloat16)
    attr16 = attr_ref[...].astype(jnp.bfloat16)
    h = jnp.dot(attr16, w1_ref[...].astype(jnp.bfloat16),
                preferred_element_type=jnp.float32)
    h = jnp.maximum(h + b1_ref[...], 0.0)
    h_aug = jnp.concatenate(
        [h.astype(jnp.bfloat16), jnp.ones((et, 1), jnp.bfloat16)], axis=1)
    p = jax.lax.dot_general(u, h_aug, (((1,), (0,)), ((), ())),
                            preferred_element_type=jnp.float32)
    acc_ref[...] += p[0:num_graphs, :] - p[1:num_graphs + 1, :]

    @pl.when(i == num_tiles - 1)
    def _finish():
        a = acc_ref[...]
        hidden = w2_ref.shape[0]
        gf = a[:, 0:hidden] / jnp.maximum(a[:, hidden:hidden + 1], 1.0)
        h2 = jnp.dot(gf, w2_ref[...], preferred_element_type=jnp.float32)
        h2 = jnp.maximum(h2 + b2_ref[...], 0.0)
        out = jnp.dot(h2, w3_ref[...], preferred_element_type=jnp.float32)
        out_ref[...] = out + b3_ref[...]


@jax.jit
def kernel(edge_index, edge_attr, batch, W1, b1, W2, b2, W3, b3):
    n_edges, in_dim = edge_attr.shape
    n_nodes = batch.shape[0]
    hidden = W1.shape[1]
    out_dim = W3.shape[1]
    num_graphs = 128

    et = _EDGE_TILE
    pad_e = (-n_edges) % et
    src = edge_index[0].astype(jnp.int32)
    attr = edge_attr
    if pad_e:
        src = jnp.concatenate(
            [src, jnp.full((pad_e,), n_nodes, jnp.int32)])
        attr = jnp.concatenate(
            [attr, jnp.zeros((pad_e, in_dim), attr.dtype)])
    num_tiles = (n_edges + pad_e) // et
    src = src.reshape(num_tiles, 1, et)

    b32 = batch.astype(jnp.int32)
    pad_n = (-n_nodes) % _NODE_CHUNK
    if pad_n:
        b32 = jnp.concatenate(
            [b32, jnp.full((pad_n,), num_graphs, jnp.int32)])
    node_rows = (n_nodes + pad_n) // _NODE_CHUNK
    b32 = b32.reshape(node_rows, _NODE_CHUNK)

    grid = (num_tiles,)
    const = lambda i: (0, 0)
    out = pl.pallas_call(
        functools.partial(_fused_kernel, num_tiles, num_graphs, n_nodes,
                          node_rows),
        grid=grid,
        in_specs=[
            pl.BlockSpec((1, 1, et), lambda i: (i, 0, 0)),
            pl.BlockSpec((et, in_dim), lambda i: (i, 0)),
            pl.BlockSpec((node_rows, _NODE_CHUNK), const),
            pl.BlockSpec((in_dim, hidden), const),
            pl.BlockSpec((1, hidden), const),
            pl.BlockSpec((hidden, hidden), const),
            pl.BlockSpec((1, hidden), const),
            pl.BlockSpec((hidden, out_dim), const),
            pl.BlockSpec((1, out_dim), const),
        ],
        out_specs=pl.BlockSpec((num_graphs, out_dim), const),
        out_shape=jax.ShapeDtypeStruct((num_graphs, out_dim), jnp.float32),
        scratch_shapes=[
            pltpu.VMEM((_NB, 1), jnp.int32),
            pltpu.VMEM((num_graphs, hidden + 1), jnp.float32),
        ],
        compiler_params=pltpu.CompilerParams(
            dimension_semantics=("arbitrary",)),
    )(src, attr, b32, W1, b1.reshape(1, hidden), W2,
      b2.reshape(1, hidden), W3, b3.reshape(1, out_dim))
    return out
